# baseline (speedup 1.0000x reference)
import functools
import jax
import jax.numpy as jnp
from jax import lax
from jax.experimental import pallas as pl
from jax.experimental.pallas import tpu as pltpu
from jax.experimental.pallas import tpu_sc as plsc

NUM_CORES = 2
NUM_SUBCORES = 16
NUM_WORKERS = NUM_CORES * NUM_SUBCORES


def _tc_add_body(x_ref, w_ref, o_ref):
    o_ref[...] = x_ref[...] + w_ref[...][None, :, :]


def _tc_add(x, W, bb=128):
    batch, seq_len, emb = x.shape
    return pl.pallas_call(
        _tc_add_body,
        grid=(batch // bb,),
        in_specs=[
            pl.BlockSpec((bb, seq_len, emb), lambda i: (i, 0, 0)),
            pl.BlockSpec((seq_len, emb), lambda i: (0, 0)),
        ],
        out_specs=pl.BlockSpec((bb, seq_len, emb), lambda i: (i, 0, 0)),
        out_shape=jax.ShapeDtypeStruct((batch, seq_len, emb), x.dtype),
        compiler_params=pltpu.CompilerParams(
            dimension_semantics=("parallel",),
        ),
    )(x, W)


def _sc_add(x, W):
    batch, seq_len, emb = x.shape
    row_words = seq_len * emb
    rows_per_w = batch // NUM_WORKERS
    x_flat = x.reshape(batch, row_words)
    w_flat = W.reshape(-1, row_words)
    mesh = plsc.VectorSubcoreMesh(core_axis_name="c", subcore_axis_name="s")

    unroll = 16
    chunk = 16 * unroll

    @functools.partial(
        pl.kernel,
        mesh=mesh,
        out_type=jax.ShapeDtypeStruct((batch, row_words), jnp.float32),
        scratch_types=[
            pltpu.VMEM((row_words,), jnp.float32),
            pltpu.VMEM((row_words,), jnp.float32),
            pltpu.VMEM((row_words,), jnp.float32),
            pltpu.VMEM((row_words,), jnp.float32),
            pltpu.VMEM((row_words,), jnp.float32),
            pltpu.SemaphoreType.DMA,
            pltpu.SemaphoreType.DMA,
        ],
    )
    def body(x_hbm, w_hbm, o_hbm, w_v, buf0, buf1, buf2, buf3, sem_in, sem_out):
        wid = lax.axis_index("s") * NUM_CORES + lax.axis_index("c")
        base = wid * rows_per_w
        bufs = (buf0, buf1, buf2, buf3)
        nbuf = len(bufs)
        pltpu.sync_copy(w_hbm.at[0], w_v)

        def add_row(buf):
            def chunk_body(j, c2):
                off = j * chunk
                for k in range(unroll):
                    o2 = off + k * 16
                    buf[pl.ds(o2, 16)] = buf[pl.ds(o2, 16)] + w_v[pl.ds(o2, 16)]
                return c2

            lax.fori_loop(0, row_words // chunk, chunk_body, 0)

        h_in = [None] * rows_per_w
        h_out = [None] * rows_per_w
        h_in[0] = pltpu.async_copy(x_hbm.at[base], bufs[0], sem_in)
        h_in[1] = pltpu.async_copy(x_hbm.at[base + 1], bufs[1], sem_in)
        for r in range(rows_per_w):
            b = r % nbuf
            if r + 2 < rows_per_w:
                if r >= 2:
                    h_out[r - 2].wait()
                h_in[r + 2] = pltpu.async_copy(
                    x_hbm.at[base + r + 2], bufs[(r + 2) % nbuf], sem_in)
            h_in[r].wait()
            add_row(bufs[b])
            h_out[r] = pltpu.async_copy(bufs[b], o_hbm.at[base + r], sem_out)
        for r in range(rows_per_w - 4, rows_per_w):
            h_out[r].wait()

    out = body(x_flat, w_flat)
    return out.reshape(batch, seq_len, emb)


def kernel(input_words, W):
    return _sc_add(input_words, W)


# speedup vs baseline: 1.2962x; 1.2962x over previous
import functools
import jax
import jax.numpy as jnp
from jax import lax
from jax.experimental import pallas as pl
from jax.experimental.pallas import tpu as pltpu
from jax.experimental.pallas import tpu_sc as plsc

NUM_CORES = 2
NUM_SUBCORES = 16
NUM_WORKERS = NUM_CORES * NUM_SUBCORES

SC_ROWS = 192
TC_BB = 64


def _tc_add_body(x_ref, w_ref, o_ref):
    o_ref[...] = x_ref[...] + w_ref[...][None, :, :]


def _tc_add_tail(x, W):
    batch, seq_len, emb = x.shape
    off = SC_ROWS // TC_BB
    return pl.pallas_call(
        _tc_add_body,
        grid=((batch - SC_ROWS) // TC_BB,),
        in_specs=[
            pl.BlockSpec((TC_BB, seq_len, emb), lambda i: (i + off, 0, 0)),
            pl.BlockSpec((seq_len, emb), lambda i: (0, 0)),
        ],
        out_specs=pl.BlockSpec((TC_BB, seq_len, emb), lambda i: (i + off, 0, 0)),
        out_shape=jax.ShapeDtypeStruct((batch, seq_len, emb), x.dtype),
        compiler_params=pltpu.CompilerParams(
            dimension_semantics=("parallel",),
        ),
    )(x, W)


def _patch_body(full_ref, sc_ref, o_ref):
    o_ref[...] = sc_ref[...]


def _patch(tc_out, sc_out):
    batch, seq_len, emb = tc_out.shape
    pb = 96
    return pl.pallas_call(
        _patch_body,
        grid=(SC_ROWS // pb,),
        in_specs=[
            pl.BlockSpec(memory_space=pl.ANY),
            pl.BlockSpec((pb, seq_len, emb), lambda i: (i, 0, 0)),
        ],
        out_specs=pl.BlockSpec((pb, seq_len, emb), lambda i: (i, 0, 0)),
        out_shape=jax.ShapeDtypeStruct((batch, seq_len, emb), tc_out.dtype),
        input_output_aliases={0: 0},
    )(tc_out, sc_out)


def _sc_add(x, W, n_rows):
    batch, seq_len, emb = x.shape
    row_words = seq_len * emb
    rows_per_w = n_rows // NUM_WORKERS
    x_flat = x.reshape(batch, row_words)
    w_flat = W.reshape(-1, row_words)
    mesh = plsc.VectorSubcoreMesh(core_axis_name="c", subcore_axis_name="s")

    unroll = 16
    chunk = 16 * unroll

    @functools.partial(
        pl.kernel,
        mesh=mesh,
        out_type=jax.ShapeDtypeStruct((n_rows, row_words), jnp.float32),
        scratch_types=[
            pltpu.VMEM((row_words,), jnp.float32),
            pltpu.VMEM((row_words,), jnp.float32),
            pltpu.VMEM((row_words,), jnp.float32),
            pltpu.VMEM((row_words,), jnp.float32),
            pltpu.VMEM((row_words,), jnp.float32),
            pltpu.SemaphoreType.DMA,
            pltpu.SemaphoreType.DMA,
        ],
    )
    def body(x_hbm, w_hbm, o_hbm, w_v, buf0, buf1, buf2, buf3, sem_in, sem_out):
        wid = lax.axis_index("s") * NUM_CORES + lax.axis_index("c")
        base = wid * rows_per_w
        bufs = (buf0, buf1, buf2, buf3)
        nbuf = len(bufs)
        pltpu.sync_copy(w_hbm.at[0], w_v)

        def add_row(buf):
            def chunk_body(j, c2):
                off = j * chunk
                for k in range(unroll):
                    o2 = off + k * 16
                    buf[pl.ds(o2, 16)] = buf[pl.ds(o2, 16)] + w_v[pl.ds(o2, 16)]
                return c2

            lax.fori_loop(0, row_words // chunk, chunk_body, 0)

        h_in = [None] * rows_per_w
        h_out = [None] * rows_per_w
        h_in[0] = pltpu.async_copy(x_hbm.at[base], bufs[0], sem_in)
        if rows_per_w > 1:
            h_in[1] = pltpu.async_copy(x_hbm.at[base + 1], bufs[1], sem_in)
        for r in range(rows_per_w):
            b = r % nbuf
            if r + 2 < rows_per_w:
                if r >= 2:
                    h_out[r - 2].wait()
                h_in[r + 2] = pltpu.async_copy(
                    x_hbm.at[base + r + 2], bufs[(r + 2) % nbuf], sem_in)
            h_in[r].wait()
            add_row(bufs[b])
            h_out[r] = pltpu.async_copy(bufs[b], o_hbm.at[base + r], sem_out)
        for r in range(max(0, rows_per_w - 4), rows_per_w):
            h_out[r].wait()

    out = body(x_flat, w_flat)
    return out.reshape(n_rows, seq_len, emb)


def kernel(input_words, W):
    sc_out = _sc_add(input_words, W, SC_ROWS)
    tc_out = _tc_add_tail(input_words, W)
    return _patch(tc_out, sc_out)


# speedup vs baseline: 2.9251x; 2.2567x over previous
import functools
import jax
import jax.numpy as jnp
from jax import lax
from jax.experimental import pallas as pl
from jax.experimental.pallas import tpu as pltpu
from jax.experimental.pallas import tpu_sc as plsc

NUM_CORES = 2
NUM_SUBCORES = 16
NUM_WORKERS = NUM_CORES * NUM_SUBCORES

SC_ROWS = 192
TC_BB = 64


def _tc_add_body(x_ref, w_ref, o_ref):
    o_ref[...] = x_ref[...] + w_ref[...][None, :, :]


def _tc_add_tail(x, W):
    batch, seq_len, emb = x.shape
    off = SC_ROWS // TC_BB
    return pl.pallas_call(
        _tc_add_body,
        grid=((batch - SC_ROWS) // TC_BB,),
        in_specs=[
            pl.BlockSpec((TC_BB, seq_len, emb), lambda i: (i + off, 0, 0)),
            pl.BlockSpec((seq_len, emb), lambda i: (0, 0)),
        ],
        out_specs=pl.BlockSpec((TC_BB, seq_len, emb), lambda i: (i + off, 0, 0)),
        out_shape=jax.ShapeDtypeStruct((batch, seq_len, emb), x.dtype),
        compiler_params=pltpu.CompilerParams(
            dimension_semantics=("parallel",),
        ),
    )(x, W)


def _patch_body(full_ref, sc_ref, o_ref):
    o_ref[...] = sc_ref[...]


def _patch(tc_out, sc_out):
    batch, seq_len, emb = tc_out.shape
    pb = 96
    return pl.pallas_call(
        _patch_body,
        grid=(SC_ROWS // pb,),
        in_specs=[
            pl.BlockSpec(memory_space=pl.ANY),
            pl.BlockSpec((pb, seq_len, emb), lambda i: (i, 0, 0)),
        ],
        out_specs=pl.BlockSpec((pb, seq_len, emb), lambda i: (i, 0, 0)),
        out_shape=jax.ShapeDtypeStruct((batch, seq_len, emb), tc_out.dtype),
        input_output_aliases={0: 0},
    )(tc_out, sc_out)


def _sc_add(x, W, n_rows):
    batch, seq_len, emb = x.shape
    rows_per_w = n_rows // NUM_WORKERS
    mesh = plsc.VectorSubcoreMesh(core_axis_name="c", subcore_axis_name="s")

    @functools.partial(
        pl.kernel,
        mesh=mesh,
        out_type=jax.ShapeDtypeStruct((n_rows, seq_len, emb), jnp.float32),
        scratch_types=[
            pltpu.VMEM((seq_len, emb), jnp.float32),
            pltpu.VMEM((seq_len, emb), jnp.float32),
            pltpu.VMEM((seq_len, emb), jnp.float32),
            pltpu.VMEM((seq_len, emb), jnp.float32),
            pltpu.VMEM((seq_len, emb), jnp.float32),
            pltpu.SemaphoreType.DMA,
            pltpu.SemaphoreType.DMA,
        ],
    )
    def body(x_hbm, w_hbm, o_hbm, w_v, buf0, buf1, buf2, buf3, sem_in, sem_out):
        wid = lax.axis_index("s") * NUM_CORES + lax.axis_index("c")
        base = wid * rows_per_w
        bufs = (buf0, buf1, buf2, buf3)
        nbuf = len(bufs)
        pltpu.sync_copy(w_hbm.at[pl.ds(0, seq_len)], w_v)

        def add_row(buf):
            def chunk_body(p, c2):
                for k in range(emb // 16):
                    o2 = k * 16
                    buf[p, pl.ds(o2, 16)] = (
                        buf[p, pl.ds(o2, 16)] + w_v[p, pl.ds(o2, 16)])
                return c2

            lax.fori_loop(0, seq_len, chunk_body, 0)

        h_in = [None] * rows_per_w
        h_out = [None] * rows_per_w
        h_in[0] = pltpu.async_copy(x_hbm.at[base], bufs[0], sem_in)
        if rows_per_w > 1:
            h_in[1] = pltpu.async_copy(x_hbm.at[base + 1], bufs[1], sem_in)
        for r in range(rows_per_w):
            b = r % nbuf
            if r + 2 < rows_per_w:
                if r >= 2:
                    h_out[r - 2].wait()
                h_in[r + 2] = pltpu.async_copy(
                    x_hbm.at[base + r + 2], bufs[(r + 2) % nbuf], sem_in)
            h_in[r].wait()
            add_row(bufs[b])
            h_out[r] = pltpu.async_copy(bufs[b], o_hbm.at[base + r], sem_out)
        for r in range(max(0, rows_per_w - 4), rows_per_w):
            h_out[r].wait()

    return body(x, W)


def kernel(input_words, W):
    sc_out = _sc_add(input_words, W, SC_ROWS)
    tc_out = _tc_add_tail(input_words, W)
    return _patch(tc_out, sc_out)


# speedup vs baseline: 3.1456x; 1.0754x over previous
import functools
import jax
import jax.numpy as jnp
from jax import lax
from jax.experimental import pallas as pl
from jax.experimental.pallas import tpu as pltpu
from jax.experimental.pallas import tpu_sc as plsc

NUM_CORES = 2
NUM_SUBCORES = 16
NUM_WORKERS = NUM_CORES * NUM_SUBCORES

SC_ROWS = 64
TC_BB = 64


def _tc_add_body(x_ref, w_ref, o_ref):
    o_ref[...] = x_ref[...] + w_ref[...][None, :, :]


def _tc_add_tail(x, W):
    batch, seq_len, emb = x.shape
    off = SC_ROWS // TC_BB
    return pl.pallas_call(
        _tc_add_body,
        grid=((batch - SC_ROWS) // TC_BB,),
        in_specs=[
            pl.BlockSpec((TC_BB, seq_len, emb), lambda i: (i + off, 0, 0)),
            pl.BlockSpec((seq_len, emb), lambda i: (0, 0)),
        ],
        out_specs=pl.BlockSpec((TC_BB, seq_len, emb), lambda i: (i + off, 0, 0)),
        out_shape=jax.ShapeDtypeStruct((batch, seq_len, emb), x.dtype),
        compiler_params=pltpu.CompilerParams(
            dimension_semantics=("parallel",),
        ),
    )(x, W)


def _patch_body(full_ref, sc_ref, o_ref):
    o_ref[...] = sc_ref[...]


def _patch(tc_out, sc_out):
    batch, seq_len, emb = tc_out.shape
    pb = min(SC_ROWS, 96)
    return pl.pallas_call(
        _patch_body,
        grid=(SC_ROWS // pb,),
        in_specs=[
            pl.BlockSpec(memory_space=pl.ANY),
            pl.BlockSpec((pb, seq_len, emb), lambda i: (i, 0, 0)),
        ],
        out_specs=pl.BlockSpec((pb, seq_len, emb), lambda i: (i, 0, 0)),
        out_shape=jax.ShapeDtypeStruct((batch, seq_len, emb), tc_out.dtype),
        input_output_aliases={0: 0},
    )(tc_out, sc_out)


def _sc_add(x, W, n_rows):
    batch, seq_len, emb = x.shape
    rows_per_w = n_rows // NUM_WORKERS
    mesh = plsc.VectorSubcoreMesh(core_axis_name="c", subcore_axis_name="s")

    @functools.partial(
        pl.kernel,
        mesh=mesh,
        out_type=jax.ShapeDtypeStruct((n_rows, seq_len, emb), jnp.float32),
        scratch_types=[
            pltpu.VMEM((seq_len, emb), jnp.float32),
            pltpu.VMEM((seq_len, emb), jnp.float32),
            pltpu.VMEM((seq_len, emb), jnp.float32),
            pltpu.VMEM((seq_len, emb), jnp.float32),
            pltpu.VMEM((seq_len, emb), jnp.float32),
            pltpu.SemaphoreType.DMA,
            pltpu.SemaphoreType.DMA,
        ],
    )
    def body(x_hbm, w_hbm, o_hbm, w_v, buf0, buf1, buf2, buf3, sem_in, sem_out):
        wid = lax.axis_index("s") * NUM_CORES + lax.axis_index("c")
        base = wid * rows_per_w
        bufs = (buf0, buf1, buf2, buf3)
        nbuf = len(bufs)
        pltpu.sync_copy(w_hbm.at[pl.ds(0, seq_len)], w_v)

        def add_row(buf):
            def chunk_body(p, c2):
                for k in range(emb // 16):
                    o2 = k * 16
                    buf[p, pl.ds(o2, 16)] = (
                        buf[p, pl.ds(o2, 16)] + w_v[p, pl.ds(o2, 16)])
                return c2

            lax.fori_loop(0, seq_len, chunk_body, 0)

        h_in = [None] * rows_per_w
        h_out = [None] * rows_per_w
        h_in[0] = pltpu.async_copy(x_hbm.at[base], bufs[0], sem_in)
        if rows_per_w > 1:
            h_in[1] = pltpu.async_copy(x_hbm.at[base + 1], bufs[1], sem_in)
        for r in range(rows_per_w):
            b = r % nbuf
            if r + 2 < rows_per_w:
                if r >= 2:
                    h_out[r - 2].wait()
                h_in[r + 2] = pltpu.async_copy(
                    x_hbm.at[base + r + 2], bufs[(r + 2) % nbuf], sem_in)
            h_in[r].wait()
            add_row(bufs[b])
            h_out[r] = pltpu.async_copy(bufs[b], o_hbm.at[base + r], sem_out)
        for r in range(max(0, rows_per_w - 4), rows_per_w):
            h_out[r].wait()

    return body(x, W)


def kernel(input_words, W):
    sc_out = _sc_add(input_words, W, SC_ROWS)
    tc_out = _tc_add_tail(input_words, W)
    return _patch(tc_out, sc_out)


# speedup vs baseline: 4.4426x; 1.4123x over previous
import jax
import jax.numpy as jnp
from jax.experimental import pallas as pl
from jax.experimental.pallas import tpu as pltpu


def _add_body(x_ref, w_ref, o_ref):
    o_ref[...] = x_ref[...] + w_ref[...][None, :, :]


def kernel(input_words, W):
    batch, seq_len, emb = input_words.shape
    bb = 128
    return pl.pallas_call(
        _add_body,
        grid=(batch // bb,),
        in_specs=[
            pl.BlockSpec((bb, seq_len, emb), lambda i: (i, 0, 0)),
            pl.BlockSpec((seq_len, emb), lambda i: (0, 0)),
        ],
        out_specs=pl.BlockSpec((bb, seq_len, emb), lambda i: (i, 0, 0)),
        out_shape=jax.ShapeDtypeStruct((batch, seq_len, emb), input_words.dtype),
        compiler_params=pltpu.CompilerParams(
            dimension_semantics=("parallel",),
        ),
    )(input_words, W)
